# baseline (speedup 1.0000x reference)
import functools

import jax
import jax.numpy as jnp
from jax import lax
from jax.experimental import pallas as pl
from jax.experimental.pallas import tpu as pltpu
from jax.experimental.pallas import tpu_sc as plsc

POOL = 512
K = 4
LEN = 8
DIM = 768
BATCH = 1024

NUM_WORKERS = 32
ROWS_TOTAL = BATCH * K
TC_ROWS = 2048
SC_ROWS = ROWS_TOTAL - TC_ROWS
SC_PER_W = SC_ROWS // NUM_WORKERS
CHUNK = 8
NCHUNK = SC_PER_W // CHUNK
NBUF = 2
TC_BLOCK = 128
TC_NBLK = TC_ROWS // TC_BLOCK


def _topk_body(qn_ref, knt_ref, idx_ref, loss_ref, canvas_ref):
    del canvas_ref
    d = jnp.dot(qn_ref[...], knt_ref[...],
                preferred_element_type=jnp.float32)
    lane = lax.broadcasted_iota(jnp.int32, d.shape, 1)
    loss = jnp.float32(0.0)
    for t in range(K):
        m = jnp.max(d, axis=1, keepdims=True)
        im = jnp.min(jnp.where(d == m, lane, POOL), axis=1, keepdims=True)
        idx_ref[:, t : t + 1] = im
        loss = loss + jnp.sum(jnp.abs(m))
        d = jnp.where(lane == im, -jnp.inf, d)
    loss_ref[...] = jnp.full((1, 1), loss / jnp.float32(BATCH), jnp.float32)


def _tc_topk(query, keys):
    return pl.pallas_call(
        _topk_body,
        out_shape=(
            jax.ShapeDtypeStruct((BATCH, K), jnp.int32),
            jax.ShapeDtypeStruct((1, 1), jnp.float32),
            jax.ShapeDtypeStruct((ROWS_TOTAL, LEN, DIM), jnp.float32),
        ),
        out_specs=(
            pl.BlockSpec((BATCH, K), lambda: (0, 0)),
            pl.BlockSpec((1, 1), lambda: (0, 0)),
            pl.BlockSpec(memory_space=pl.ANY),
        ),
    )(query, keys)


def _sc_gather_body(idx_hbm, table_hbm, canvas_hbm, dummy, idx_v, *bufs):
    wid = lax.axis_index("s") * 2 + lax.axis_index("c")
    rows_v = bufs[:NBUF]
    gsem = bufs[NBUF:2 * NBUF]
    ssem = bufs[2 * NBUF:]

    pltpu.sync_copy(idx_hbm.at[TC_ROWS // 128 + wid // 2], idx_v)
    ioff = (wid % 2) * SC_PER_W
    base = TC_ROWS + wid * SC_PER_W

    def fire_gather(c):
        pltpu.make_async_copy(
            table_hbm.at[idx_v.at[pl.ds(ioff + c * CHUNK, CHUNK)]],
            rows_v[c % NBUF], gsem[c % NBUF]).start()

    for c in range(min(NBUF - 1, NCHUNK)):
        fire_gather(c)
    for c in range(NCHUNK):
        b = c % NBUF
        if c + NBUF - 1 < NCHUNK:
            if c >= 1:
                bb = (c + NBUF - 1) % NBUF
                pltpu.make_async_copy(
                    rows_v[bb], canvas_hbm.at[pl.ds(base + (c - 1) * CHUNK, CHUNK)],
                    ssem[bb]).wait()
            fire_gather(c + NBUF - 1)
        pltpu.make_async_copy(
            table_hbm.at[idx_v.at[pl.ds(ioff + c * CHUNK, CHUNK)]],
            rows_v[b], gsem[b]).wait()
        pltpu.make_async_copy(
            rows_v[b], canvas_hbm.at[pl.ds(base + c * CHUNK, CHUNK)], ssem[b]).start()
    for c in range(max(NCHUNK - NBUF, 0), NCHUNK):
        pltpu.make_async_copy(
            rows_v[c % NBUF], canvas_hbm.at[pl.ds(base + c * CHUNK, CHUNK)],
            ssem[c % NBUF]).wait()
    pltpu.sync_copy(idx_v, dummy.at[wid])


@functools.cache
def _sc_gather():
    return pl.kernel(
        _sc_gather_body,
        mesh=plsc.VectorSubcoreMesh(core_axis_name="c", subcore_axis_name="s"),
        out_type=jax.ShapeDtypeStruct((NUM_WORKERS, 128), jnp.int32),
        scratch_types=(
            [pltpu.VMEM((128,), jnp.int32)]
            + [pltpu.VMEM((CHUNK, LEN, DIM), jnp.float32)] * NBUF
            + [pltpu.SemaphoreType.DMA] * (2 * NBUF)
        ),
        compiler_params=pltpu.CompilerParams(
            use_tc_tiling_on_sc=True, has_side_effects=True),
    )


def _tc_fill_body(idx_ref, table_ref, canvas_ref, dummy_ref, s0, s1, sem0, sem1):
    scr = (s0, s1)
    sem = (sem0, sem1)

    def copy_rows(blk, scratch):
        for i in range(TC_BLOCK):
            r = idx_ref[0, blk, i]
            scratch[pl.ds(i, 1)] = table_ref[pl.ds(r, 1)]

    for blk in range(TC_NBLK):
        b = blk % 2
        if blk >= 2:
            pltpu.make_async_copy(
                scr[b], canvas_ref.at[pl.ds((blk - 2) * TC_BLOCK, TC_BLOCK)],
                sem[b]).wait()
        copy_rows(blk, scr[b])
        pltpu.make_async_copy(
            scr[b], canvas_ref.at[pl.ds(blk * TC_BLOCK, TC_BLOCK)], sem[b]).start()
    for blk in (TC_NBLK - 2, TC_NBLK - 1):
        pltpu.make_async_copy(
            scr[blk % 2], canvas_ref.at[pl.ds(blk * TC_BLOCK, TC_BLOCK)],
            sem[blk % 2]).wait()
    dummy_ref[...] = jnp.zeros((8, 128), jnp.float32)


def _tc_fill(idx3, table, canvas):
    return pl.pallas_call(
        _tc_fill_body,
        in_specs=[
            pl.BlockSpec((1, ROWS_TOTAL // 128, 128), lambda: (0, 0, 0),
                         memory_space=pltpu.SMEM),
            pl.BlockSpec((POOL, LEN, DIM), lambda: (0, 0, 0)),
            pl.BlockSpec(memory_space=pl.ANY),
        ],
        out_specs=pl.BlockSpec((8, 128), lambda: (0, 0)),
        out_shape=jax.ShapeDtypeStruct((8, 128), jnp.float32),
        scratch_shapes=[
            pltpu.VMEM((TC_BLOCK, LEN, DIM), jnp.float32),
            pltpu.VMEM((TC_BLOCK, LEN, DIM), jnp.float32),
            pltpu.SemaphoreType.DMA,
            pltpu.SemaphoreType.DMA,
        ],
        compiler_params=pltpu.CompilerParams(has_side_effects=True),
    )(idx3, table, canvas)


def _l2norm(x):
    n = jnp.linalg.norm(x, axis=-1, keepdims=True)
    return x / jnp.maximum(n, 1e-12)


def kernel(query, prompt_keys, prompt_values):
    idx, loss, canvas = _tc_topk(_l2norm(query), _l2norm(prompt_keys).T)
    idx32 = idx.reshape(ROWS_TOTAL // 128, 128)
    d1 = _sc_gather()(idx32, prompt_values, canvas)
    d2 = _tc_fill(idx32.reshape(1, ROWS_TOTAL // 128, 128), prompt_values, canvas)
    rows, _, _ = lax.optimization_barrier((canvas, d1, d2))
    return rows.reshape(BATCH, K, LEN, DIM), loss.reshape(())
